# baseline (speedup 1.0000x reference)
import jax
import jax.numpy as jnp
from jax.experimental import pallas as pl
from jax.experimental.pallas import tpu as pltpu

N_WAY, N_SUPPORT, EMB = 20, 5, 128
B, NBOX = 16, 8732
R = B * NBOX
GRID = 59
CA = R // GRID
LROWS = CA * 4 // 128


def _body(t1_ref, t2_ref, cls_ref, lp_ref, lt_ref, sup_ref, out_ref,
          acc_vec, acc_smem):
    i = pl.program_id(0)

    @pl.when(i == 0)
    def _init():
        acc_vec[...] = jnp.zeros_like(acc_vec)
        acc_smem[0] = 0.0
        acc_smem[1] = 0.0

    posf = (t1_ref[0] > 0).astype(jnp.float32)
    acc_vec[...] += jax.lax.dot(posf, cls_ref[0])
    acc_smem[1] += jnp.sum(posf)

    posf2 = (t2_ref[0] > 0).astype(jnp.float32)
    lane = jax.lax.broadcasted_iota(jnp.int32, (32, 128), 1)
    grp = jax.lax.broadcasted_iota(jnp.int32, (32, 128), 0)
    expand = (lane // 4 == grp).astype(jnp.float32)
    posf4 = jax.lax.dot(posf2, expand)

    diff = lp_ref[0] - lt_ref[0]
    a = jnp.abs(diff)
    sl1 = jnp.where(a < 1.0, 0.5 * diff * diff, a - 0.5)
    acc_smem[0] += jnp.sum(sl1 * posf4)

    @pl.when(i == pl.num_programs(0) - 1)
    def _fin():
        num_pos = acc_smem[1]
        loc_loss = acc_smem[0]
        mean_q = acc_vec[...] / num_pos
        protos = (sup_ref[:, 0, :] + sup_ref[:, 1, :] + sup_ref[:, 2, :]
                  + sup_ref[:, 3, :] + sup_ref[:, 4, :]) * (1.0 / N_SUPPORT)
        d = jnp.sum((mean_q - protos) ** 2, axis=1)
        neg = -d
        m = jnp.max(neg)
        lse = m + jnp.log(jnp.sum(jnp.exp(neg - m)))
        cls_loss = lse - neg[0]
        out_ref[...] = jnp.full((1, 1), cls_loss + loc_loss / num_pos,
                                dtype=jnp.float32)


def kernel(loc_preds, loc_targets, cls_preds, cls_targets, supports):
    t_flat = cls_targets.reshape(-1)
    t1 = t_flat.reshape(GRID, 1, CA)
    t2 = t_flat.reshape(GRID, LROWS, 32)
    cls3 = cls_preds.reshape(GRID, CA, EMB)
    lp = loc_preds.reshape(GRID, LROWS, 128)
    lt = loc_targets.reshape(GRID, LROWS, 128)
    out = pl.pallas_call(
        _body,
        grid=(GRID,),
        in_specs=[
            pl.BlockSpec((1, 1, CA), lambda i: (i, 0, 0)),
            pl.BlockSpec((1, LROWS, 32), lambda i: (i, 0, 0)),
            pl.BlockSpec((1, CA, EMB), lambda i: (i, 0, 0)),
            pl.BlockSpec((1, LROWS, 128), lambda i: (i, 0, 0)),
            pl.BlockSpec((1, LROWS, 128), lambda i: (i, 0, 0)),
            pl.BlockSpec((N_WAY, N_SUPPORT, EMB), lambda i: (0, 0, 0)),
        ],
        out_specs=pl.BlockSpec((1, 1), lambda i: (0, 0)),
        out_shape=jax.ShapeDtypeStruct((1, 1), jnp.float32),
        scratch_shapes=[
            pltpu.VMEM((1, EMB), jnp.float32),
            pltpu.SMEM((2,), jnp.float32),
        ],
        compiler_params=pltpu.CompilerParams(
            dimension_semantics=("arbitrary",),
        ),
    )(t1, t2, cls3, lp, lt, supports)
    return out[0, 0]


# speedup vs baseline: 1.9730x; 1.9730x over previous
import jax
import jax.numpy as jnp
from jax.experimental import pallas as pl
from jax.experimental.pallas import tpu as pltpu

N_WAY, N_SUPPORT, EMB = 20, 5, 128
B, NBOX = 16, 8732


def _body(t_ref, cls_ref, lp_ref, lt_ref, sup_ref, out_ref, acc_vec, acc_smem):
    i = pl.program_id(0)

    @pl.when(i == 0)
    def _init():
        acc_vec[...] = jnp.zeros_like(acc_vec)
        acc_smem[0] = 0.0
        acc_smem[1] = 0.0

    posf = (t_ref[0] > 0).astype(jnp.float32)
    acc_vec[...] += jax.lax.dot(posf, cls_ref[0])
    acc_smem[1] += jnp.sum(posf)

    diff = lp_ref[0] - lt_ref[0]
    a = jnp.abs(diff)
    sl1 = jnp.where(a < 1.0, 0.5 * diff * diff, a - 0.5)
    acc_smem[0] += jnp.sum(jax.lax.dot(posf, sl1))

    @pl.when(i == pl.num_programs(0) - 1)
    def _fin():
        num_pos = acc_smem[1]
        loc_loss = acc_smem[0]
        mean_q = acc_vec[...] / num_pos
        protos = (sup_ref[:, 0, :] + sup_ref[:, 1, :] + sup_ref[:, 2, :]
                  + sup_ref[:, 3, :] + sup_ref[:, 4, :]) * (1.0 / N_SUPPORT)
        d = jnp.sum((mean_q - protos) ** 2, axis=1)
        neg = -d
        m = jnp.max(neg)
        lse = m + jnp.log(jnp.sum(jnp.exp(neg - m)))
        cls_loss = lse - neg[0]
        out_ref[...] = jnp.full((1, 1), cls_loss + loc_loss / num_pos,
                                dtype=jnp.float32)


def kernel(loc_preds, loc_targets, cls_preds, cls_targets, supports):
    t3 = cls_targets.reshape(B, 1, NBOX)
    out = pl.pallas_call(
        _body,
        grid=(B,),
        in_specs=[
            pl.BlockSpec((1, 1, NBOX), lambda i: (i, 0, 0)),
            pl.BlockSpec((1, NBOX, EMB), lambda i: (i, 0, 0)),
            pl.BlockSpec((1, NBOX, 4), lambda i: (i, 0, 0)),
            pl.BlockSpec((1, NBOX, 4), lambda i: (i, 0, 0)),
            pl.BlockSpec((N_WAY, N_SUPPORT, EMB), lambda i: (0, 0, 0)),
        ],
        out_specs=pl.BlockSpec((1, 1), lambda i: (0, 0)),
        out_shape=jax.ShapeDtypeStruct((1, 1), jnp.float32),
        scratch_shapes=[
            pltpu.VMEM((1, EMB), jnp.float32),
            pltpu.SMEM((2,), jnp.float32),
        ],
        compiler_params=pltpu.CompilerParams(
            dimension_semantics=("arbitrary",),
        ),
    )(t3, cls_preds, loc_preds, loc_targets, supports)
    return out[0, 0]
